# Initial kernel scaffold; baseline (speedup 1.0000x reference)
import jax
import jax.numpy as jnp
from jax.experimental import pallas as pl


def kernel(x_batch, y_batch, W1, b1, W2, b2, W3, b3, centers):
    raise NotImplementedError("write your pallas kernel here")


# speedup vs baseline: 1.4706x; 1.4706x over previous
import functools

import jax
import jax.numpy as jnp
from jax import lax
from jax.experimental import pallas as pl
from jax.experimental.pallas import tpu as pltpu
from jax.experimental.pallas import tpu_sc as plsc

B = 16384
XD = 512
YD = 64
NK = 64
HID = 2048
TB = 256
NB = B // TB
NWI = NB + NK - 1

_NC = 2
_NS = 16
_NW = _NC * _NS
_BPW = B // _NW
_CH = 128
_NCH = _BPW // _CH


def _labels_body(y_ref, c_ref, lab_ref):
    y = y_ref[...]
    c = c_ref[...]
    d2 = (jnp.sum(y * y, axis=1, keepdims=True)
          - 2.0 * jax.lax.dot_general(y, c, (((1,), (1,)), ((), ())),
                                      preferred_element_type=jnp.float32)
          + jnp.sum(c * c, axis=1)[None, :])
    lab = jnp.argmin(d2, axis=1).astype(jnp.int32)
    lab_ref[...] = jnp.broadcast_to(lab[:, None], (TB, NK))


def _mlp_body(xcat_ref, w1_ref, b1_ref, w2_ref, b2_ref, w3l_ref, b3l_ref,
              ohct_ref, h2_ref, t_ref, ce_ref):
    xs = xcat_ref[:, :XD].astype(jnp.bfloat16)
    h1 = jnp.dot(xs, w1_ref[...], preferred_element_type=jnp.float32)
    h1 = jnp.maximum(h1 + b1_ref[...], 0.0).astype(jnp.bfloat16)
    h2 = jnp.dot(h1, w2_ref[...], preferred_element_type=jnp.float32)
    h2 = jnp.maximum(h2 + b2_ref[...], 0.0)
    h2b = h2.astype(jnp.bfloat16)
    h2_ref[...] = h2b
    logits = jnp.dot(h2b, w3l_ref[...], preferred_element_type=jnp.float32)
    logits = logits + b3l_ref[...]
    m = jnp.max(logits, axis=1, keepdims=True)
    lse = m[:, 0] + jnp.log(jnp.sum(jnp.exp(logits - m), axis=1))
    picked = jnp.sum(logits * ohct_ref[:, :NK], axis=1)
    ce_part = jnp.sum(lse - picked)
    t_ref[...] = xcat_ref[:, XD:XD + YD] - ohct_ref[:, NK:]
    i = pl.program_id(0)
    prev = jnp.where(i == 0, jnp.zeros((1, 1), jnp.float32), ce_ref[...])
    ce_ref[...] = prev + ce_part


def _res_body(bids_ref, eids_ref, vflg_ref, h2_ref, w3r_ref, b3r_ref,
              t_ref, ohct_ref, mse_ref):
    j = pl.program_id(0)
    e = eids_ref[j]
    v = vflg_ref[j]
    p = jnp.dot(h2_ref[...], w3r_ref[0], preferred_element_type=jnp.float32)
    p = p + b3r_ref[0]
    lane = lax.broadcasted_iota(jnp.int32, (TB, NK), 1)
    sel = jnp.where(lane == e, ohct_ref[:, :NK], 0.0)
    rs = jnp.sum(sel, axis=1, keepdims=True)
    d = t_ref[...] - p
    contrib = jnp.sum(d * d * rs) * v.astype(jnp.float32)
    prev = jnp.where(j == 0, jnp.zeros((1, 1), jnp.float32), mse_ref[...])
    mse_ref[...] = prev + contrib


def _sc_gather_body(t1_hbm, t2_hbm, sidx_hbm, slab_hbm,
                    g1_hbm, g2_hbm,
                    idx_v, lab_v, buf1, buf2, s1, s2):
    wid = lax.axis_index("s") * _NC + lax.axis_index("c")
    base = wid * _BPW

    def chunk(ci, carry):
        off = base + ci * _CH
        pltpu.sync_copy(sidx_hbm.at[pl.ds(off, _CH)], idx_v)
        pltpu.sync_copy(slab_hbm.at[pl.ds(off, _CH)], lab_v)
        c1 = pltpu.async_copy(t1_hbm.at[idx_v], buf1, s1)
        c2 = pltpu.async_copy(t2_hbm.at[lab_v], buf2, s2)
        c1.wait()
        c2.wait()
        pltpu.sync_copy(buf1, g1_hbm.at[pl.ds(off, _CH)])
        pltpu.sync_copy(buf2, g2_hbm.at[pl.ds(off, _CH)])
        return carry

    lax.fori_loop(0, _NCH, chunk, 0)


def _sc_gather(t1, t2, sidx, slab):
    mesh = plsc.VectorSubcoreMesh(core_axis_name="c", subcore_axis_name="s",
                                  num_cores=_NC, num_subcores=_NS)
    f = pl.kernel(
        _sc_gather_body,
        out_type=[
            jax.ShapeDtypeStruct((B, XD + 2 * YD), jnp.float32),
            jax.ShapeDtypeStruct((B, NK + YD), jnp.float32),
        ],
        mesh=mesh,
        scratch_types=[
            pltpu.VMEM((_CH,), jnp.int32),
            pltpu.VMEM((_CH,), jnp.int32),
            pltpu.VMEM((_CH, XD + 2 * YD), jnp.float32),
            pltpu.VMEM((_CH, NK + YD), jnp.float32),
            pltpu.SemaphoreType.DMA,
            pltpu.SemaphoreType.DMA,
        ],
    )
    return f(t1, t2, sidx, slab)


def _compute_labels(y_batch, centers):
    return pl.pallas_call(
        _labels_body,
        grid=(NB,),
        in_specs=[
            pl.BlockSpec((TB, YD), lambda i: (i, 0)),
            pl.BlockSpec((NK, YD), lambda i: (0, 0)),
        ],
        out_specs=pl.BlockSpec((TB, NK), lambda i: (i, 0)),
        out_shape=jax.ShapeDtypeStruct((B, NK), jnp.int32),
    )(y_batch, centers)


def _mlp_ce(g1, w1x, b1, w2, b2, w3l, b3l, g2):
    return pl.pallas_call(
        _mlp_body,
        grid=(NB,),
        in_specs=[
            pl.BlockSpec((TB, XD + 2 * YD), lambda i: (i, 0)),
            pl.BlockSpec((XD, HID), lambda i: (0, 0)),
            pl.BlockSpec((1, HID), lambda i: (0, 0)),
            pl.BlockSpec((HID, HID), lambda i: (0, 0)),
            pl.BlockSpec((1, HID), lambda i: (0, 0)),
            pl.BlockSpec((HID, NK), lambda i: (0, 0)),
            pl.BlockSpec((1, NK), lambda i: (0, 0)),
            pl.BlockSpec((TB, NK + YD), lambda i: (i, 0)),
        ],
        out_specs=[
            pl.BlockSpec((TB, HID), lambda i: (i, 0)),
            pl.BlockSpec((TB, YD), lambda i: (i, 0)),
            pl.BlockSpec((1, 1), lambda i: (0, 0)),
        ],
        out_shape=[
            jax.ShapeDtypeStruct((B, HID), jnp.bfloat16),
            jax.ShapeDtypeStruct((B, YD), jnp.float32),
            jax.ShapeDtypeStruct((1, 1), jnp.float32),
        ],
    )(g1, w1x, b1, w2, b2, w3l, b3l, g2)


def _residual_mse(bids, eids, vflg, h2, w3r, b3r, t, g2):
    grid_spec = pltpu.PrefetchScalarGridSpec(
        num_scalar_prefetch=3,
        grid=(NWI,),
        in_specs=[
            pl.BlockSpec((TB, HID), lambda j, bids, eids, vflg: (bids[j], 0)),
            pl.BlockSpec((1, HID, YD), lambda j, bids, eids, vflg: (eids[j], 0, 0)),
            pl.BlockSpec((1, 1, YD), lambda j, bids, eids, vflg: (eids[j], 0, 0)),
            pl.BlockSpec((TB, YD), lambda j, bids, eids, vflg: (bids[j], 0)),
            pl.BlockSpec((TB, NK + YD), lambda j, bids, eids, vflg: (bids[j], 0)),
        ],
        out_specs=pl.BlockSpec((1, 1), lambda j, bids, eids, vflg: (0, 0)),
    )
    return pl.pallas_call(
        _res_body,
        grid_spec=grid_spec,
        out_shape=jax.ShapeDtypeStruct((1, 1), jnp.float32),
    )(bids, eids, vflg, h2, w3r, b3r, t, g2)


def kernel(x_batch, y_batch, W1, b1, W2, b2, W3, b3, centers):
    lab_full = _compute_labels(y_batch, centers)
    labels = lab_full[:, 0]

    slab, sidx = lax.sort_key_val(labels, jnp.arange(B, dtype=jnp.int32))
    blk = slab.reshape(NB, TB)
    lo = blk[:, 0]
    hi = blk[:, -1]
    nb = (hi - lo + 1).astype(jnp.int32)
    base = jnp.concatenate([jnp.zeros((1,), jnp.int32), jnp.cumsum(nb)])
    jv = jnp.arange(NWI, dtype=jnp.int32)
    bids = jnp.clip(jnp.searchsorted(base, jv, side="right").astype(jnp.int32) - 1,
                    0, NB - 1)
    eids = lo[bids] + (jv - base[bids])
    vflg = (jv < base[NB]).astype(jnp.int32)
    eids = jnp.where(vflg == 1, eids, 0)

    eye = jnp.eye(NK, dtype=jnp.float32)
    t1 = jnp.concatenate(
        [x_batch, y_batch, jnp.zeros((B, YD), jnp.float32)], axis=1)
    t2 = jnp.concatenate([eye, centers], axis=1)
    g1, g2 = _sc_gather(t1, t2, sidx, slab)

    w1x = W1[YD:YD + XD, :].astype(jnp.bfloat16)
    w2 = W2.astype(jnp.bfloat16)
    w3l = W3[:, :NK].astype(jnp.bfloat16)
    w3r = W3[:, NK:].reshape(HID, NK, YD).transpose(1, 0, 2).astype(jnp.bfloat16)
    b3r = b3[NK:].reshape(NK, 1, YD)
    h2, t, ce_sum = _mlp_ce(g1, w1x, b1.reshape(1, HID), w2, b2.reshape(1, HID),
                            w3l, b3[:NK].reshape(1, NK), g2)

    mse_sum = _residual_mse(bids, eids, vflg, h2, w3r, b3r, t, g2)

    return ce_sum[0, 0] / B + 100.0 * mse_sum[0, 0] / (B * YD)


# speedup vs baseline: 1.8937x; 1.2877x over previous
import jax
import jax.numpy as jnp
from jax import lax
from jax.experimental import pallas as pl
from jax.experimental.pallas import tpu as pltpu
from jax.experimental.pallas import tpu_sc as plsc

B = 16384
XD = 512
YD = 64
NK = 64
HID = 2048
TB = 256
NB = B // TB
NWI = NB + NK - 1

_NC = 2
_NS = 16
_NW = _NC * _NS
_BPW = B // _NW
_CH = 64
_NCH = _BPW // _CH


def _labels_body(y_ref, c_ref, lab_ref, toh_ref):
    y = y_ref[...]
    c = c_ref[...]
    d2 = (jnp.sum(y * y, axis=1, keepdims=True)
          - 2.0 * jax.lax.dot_general(y, c, (((1,), (1,)), ((), ())),
                                      preferred_element_type=jnp.float32)
          + jnp.sum(c * c, axis=1)[None, :])
    lab = jnp.argmin(d2, axis=1).astype(jnp.int32)
    lab_ref[...] = jnp.broadcast_to(lab[:, None], (TB, NK))
    lane = lax.broadcasted_iota(jnp.int32, (TB, NK), 1)
    oh = (lane == lab[:, None]).astype(jnp.float32)
    ct = jnp.dot(oh, c, preferred_element_type=jnp.float32)
    toh_ref[...] = jnp.concatenate([y - ct, oh], axis=1)


def _mlp_body(xs_ref, w1_ref, b1_ref, w2_ref, b2_ref, w3l_ref, b3l_ref,
              toh_ref, h2_ref, ce_ref):
    xs = xs_ref[...].astype(jnp.bfloat16)
    h1 = jnp.dot(xs, w1_ref[...], preferred_element_type=jnp.float32)
    h1 = jnp.maximum(h1 + b1_ref[...], 0.0).astype(jnp.bfloat16)
    h2 = jnp.dot(h1, w2_ref[...], preferred_element_type=jnp.float32)
    h2 = jnp.maximum(h2 + b2_ref[...], 0.0)
    h2b = h2.astype(jnp.bfloat16)
    h2_ref[...] = h2b
    logits = jnp.dot(h2b, w3l_ref[...], preferred_element_type=jnp.float32)
    logits = logits + b3l_ref[...]
    m = jnp.max(logits, axis=1, keepdims=True)
    lse = m[:, 0] + jnp.log(jnp.sum(jnp.exp(logits - m), axis=1))
    picked = jnp.sum(logits * toh_ref[:, YD:], axis=1)
    ce_part = jnp.sum(lse - picked)
    i = pl.program_id(0)
    prev = jnp.where(i == 0, jnp.zeros((1, 1), jnp.float32), ce_ref[...])
    ce_ref[...] = prev + ce_part


def _res_body(bids_ref, eids_ref, vflg_ref, h2_ref, w3r_ref, b3r_ref,
              toh_ref, mse_ref):
    j = pl.program_id(0)
    e = eids_ref[j]
    v = vflg_ref[j]
    p = jnp.dot(h2_ref[...], w3r_ref[0], preferred_element_type=jnp.float32)
    p = p + b3r_ref[0]
    lane = lax.broadcasted_iota(jnp.int32, (TB, NK), 1)
    sel = jnp.where(lane == e, toh_ref[:, YD:], 0.0)
    rs = jnp.sum(sel, axis=1, keepdims=True)
    d = toh_ref[:, :YD] - p
    contrib = jnp.sum(d * d * rs) * v.astype(jnp.float32)
    prev = jnp.where(j == 0, jnp.zeros((1, 1), jnp.float32), mse_ref[...])
    mse_ref[...] = prev + contrib


def _sc_gather_body(x_hbm, toh_hbm, sidx_hbm,
                    g1_hbm, g2_hbm,
                    idx_v, b1a, b1b, b2a, b2b, sga, sgb, swa, swb):
    wid = lax.axis_index("s") * _NC + lax.axis_index("c")
    base = wid * _BPW
    pltpu.sync_copy(sidx_hbm.at[pl.ds(base, _BPW)], idx_v)

    bufs1 = (b1a, b1b)
    bufs2 = (b2a, b2b)
    gsems = (sga, sgb)
    wsems = (swa, swb)
    writes = [None, None]
    for ci in range(_NCH):
        bi = ci % 2
        if writes[bi] is not None:
            for w in writes[bi]:
                w.wait()
        idx_c = idx_v.at[pl.ds(ci * _CH, _CH)]
        c1 = pltpu.async_copy(x_hbm.at[idx_c], bufs1[bi], gsems[bi])
        c2 = pltpu.async_copy(toh_hbm.at[idx_c], bufs2[bi], gsems[bi])
        c1.wait()
        c2.wait()
        dst = pl.ds(base + ci * _CH, _CH)
        w1 = pltpu.async_copy(bufs1[bi], g1_hbm.at[dst], wsems[bi])
        w2 = pltpu.async_copy(bufs2[bi], g2_hbm.at[dst], wsems[bi])
        writes[bi] = (w1, w2)
    for ws in writes:
        if ws is not None:
            for w in ws:
                w.wait()


def _sc_gather(x_batch, toh, sidx):
    mesh = plsc.VectorSubcoreMesh(core_axis_name="c", subcore_axis_name="s",
                                  num_cores=_NC, num_subcores=_NS)
    f = pl.kernel(
        _sc_gather_body,
        out_type=[
            jax.ShapeDtypeStruct((B, XD), jnp.float32),
            jax.ShapeDtypeStruct((B, YD + NK), jnp.float32),
        ],
        mesh=mesh,
        scratch_types=[
            pltpu.VMEM((_BPW,), jnp.int32),
            pltpu.VMEM((_CH, XD), jnp.float32),
            pltpu.VMEM((_CH, XD), jnp.float32),
            pltpu.VMEM((_CH, YD + NK), jnp.float32),
            pltpu.VMEM((_CH, YD + NK), jnp.float32),
            pltpu.SemaphoreType.DMA,
            pltpu.SemaphoreType.DMA,
            pltpu.SemaphoreType.DMA,
            pltpu.SemaphoreType.DMA,
        ],
    )
    return f(x_batch, toh, sidx)


def _compute_labels(y_batch, centers):
    return pl.pallas_call(
        _labels_body,
        grid=(NB,),
        in_specs=[
            pl.BlockSpec((TB, YD), lambda i: (i, 0)),
            pl.BlockSpec((NK, YD), lambda i: (0, 0)),
        ],
        out_specs=[
            pl.BlockSpec((TB, NK), lambda i: (i, 0)),
            pl.BlockSpec((TB, YD + NK), lambda i: (i, 0)),
        ],
        out_shape=[
            jax.ShapeDtypeStruct((B, NK), jnp.int32),
            jax.ShapeDtypeStruct((B, YD + NK), jnp.float32),
        ],
    )(y_batch, centers)


def _mlp_ce(g1, w1x, b1, w2, b2, w3l, b3l, g2):
    return pl.pallas_call(
        _mlp_body,
        grid=(NB,),
        in_specs=[
            pl.BlockSpec((TB, XD), lambda i: (i, 0)),
            pl.BlockSpec((XD, HID), lambda i: (0, 0)),
            pl.BlockSpec((1, HID), lambda i: (0, 0)),
            pl.BlockSpec((HID, HID), lambda i: (0, 0)),
            pl.BlockSpec((1, HID), lambda i: (0, 0)),
            pl.BlockSpec((HID, NK), lambda i: (0, 0)),
            pl.BlockSpec((1, NK), lambda i: (0, 0)),
            pl.BlockSpec((TB, YD + NK), lambda i: (i, 0)),
        ],
        out_specs=[
            pl.BlockSpec((TB, HID), lambda i: (i, 0)),
            pl.BlockSpec((1, 1), lambda i: (0, 0)),
        ],
        out_shape=[
            jax.ShapeDtypeStruct((B, HID), jnp.bfloat16),
            jax.ShapeDtypeStruct((1, 1), jnp.float32),
        ],
    )(g1, w1x, b1, w2, b2, w3l, b3l, g2)


def _residual_mse(bids, eids, vflg, h2, w3r, b3r, g2):
    grid_spec = pltpu.PrefetchScalarGridSpec(
        num_scalar_prefetch=3,
        grid=(NWI,),
        in_specs=[
            pl.BlockSpec((TB, HID), lambda j, bids, eids, vflg: (bids[j], 0)),
            pl.BlockSpec((1, HID, YD), lambda j, bids, eids, vflg: (eids[j], 0, 0)),
            pl.BlockSpec((1, 1, YD), lambda j, bids, eids, vflg: (eids[j], 0, 0)),
            pl.BlockSpec((TB, YD + NK), lambda j, bids, eids, vflg: (bids[j], 0)),
        ],
        out_specs=pl.BlockSpec((1, 1), lambda j, bids, eids, vflg: (0, 0)),
    )
    return pl.pallas_call(
        _res_body,
        grid_spec=grid_spec,
        out_shape=jax.ShapeDtypeStruct((1, 1), jnp.float32),
    )(bids, eids, vflg, h2, w3r, b3r, g2)


def kernel(x_batch, y_batch, W1, b1, W2, b2, W3, b3, centers):
    lab_full, toh = _compute_labels(y_batch, centers)
    labels = lab_full[:, 0]

    slab, sidx = lax.sort_key_val(labels, jnp.arange(B, dtype=jnp.int32))
    blk = slab.reshape(NB, TB)
    lo = blk[:, 0]
    hi = blk[:, -1]
    nb = (hi - lo + 1).astype(jnp.int32)
    base = jnp.concatenate([jnp.zeros((1,), jnp.int32), jnp.cumsum(nb)])
    jv = jnp.arange(NWI, dtype=jnp.int32)
    bids = jnp.clip(jnp.searchsorted(base, jv, side="right").astype(jnp.int32) - 1,
                    0, NB - 1)
    eids = lo[bids] + (jv - base[bids])
    vflg = (jv < base[NB]).astype(jnp.int32)
    eids = jnp.where(vflg == 1, eids, 0)

    g1, g2 = _sc_gather(x_batch, toh, sidx)

    w1x = W1[YD:YD + XD, :].astype(jnp.bfloat16)
    w2 = W2.astype(jnp.bfloat16)
    w3l = W3[:, :NK].astype(jnp.bfloat16)
    w3r = W3[:, NK:].reshape(HID, NK, YD).transpose(1, 0, 2).astype(jnp.bfloat16)
    b3r = b3[NK:].reshape(NK, 1, YD)
    h2, ce_sum = _mlp_ce(g1, w1x, b1.reshape(1, HID), w2, b2.reshape(1, HID),
                         w3l, b3[:NK].reshape(1, NK), g2)

    mse_sum = _residual_mse(bids, eids, vflg, h2, w3r, b3r, g2)

    return ce_sum[0, 0] / B + 100.0 * mse_sum[0, 0] / (B * YD)


# speedup vs baseline: 2.0467x; 1.0808x over previous
import jax
import jax.numpy as jnp
from jax import lax
from jax.experimental import pallas as pl
from jax.experimental.pallas import tpu as pltpu
from jax.experimental.pallas import tpu_sc as plsc

B = 16384
XD = 512
YD = 64
NK = 64
HID = 2048
NP = NK // 2
TB = 256
NB = B // TB

_NC = 2
_NS = 16
_NW = _NC * _NS
_BPW = B // _NW
_CH = 64
_NCH = _BPW // _CH


def _labels_body(y_ref, c_ref, lab_ref, toh_ref):
    y = y_ref[...]
    c = c_ref[...]
    d2 = (jnp.sum(y * y, axis=1, keepdims=True)
          - 2.0 * jax.lax.dot_general(y, c, (((1,), (1,)), ((), ())),
                                      preferred_element_type=jnp.float32)
          + jnp.sum(c * c, axis=1)[None, :])
    lab = jnp.argmin(d2, axis=1).astype(jnp.int32)
    lab_ref[...] = jnp.broadcast_to(lab[:, None], (TB, NK))
    lane = lax.broadcasted_iota(jnp.int32, (TB, NK), 1)
    oh = (lane == lab[:, None]).astype(jnp.float32)
    ct = jnp.dot(oh, c, preferred_element_type=jnp.float32)
    toh_ref[...] = jnp.concatenate([y - ct, oh], axis=1)


def _fused_body(plo_ref, pcnt_ref, xs_ref, w1_ref, b1_ref, w2_ref, b2_ref,
                w3l_ref, b3l_ref, w3r_ref, b3r_ref, toh_ref, acc_ref):
    i = pl.program_id(0)
    xs = xs_ref[...].astype(jnp.bfloat16)
    h1 = jnp.dot(xs, w1_ref[...], preferred_element_type=jnp.float32)
    h1 = jnp.maximum(h1 + b1_ref[...], 0.0).astype(jnp.bfloat16)
    h2 = jnp.dot(h1, w2_ref[...], preferred_element_type=jnp.float32)
    h2 = jnp.maximum(h2 + b2_ref[...], 0.0)
    h2b = h2.astype(jnp.bfloat16)
    logits = jnp.dot(h2b, w3l_ref[...], preferred_element_type=jnp.float32)
    logits = logits + b3l_ref[...]
    m = jnp.max(logits, axis=1, keepdims=True)
    lse = m[:, 0] + jnp.log(jnp.sum(jnp.exp(logits - m), axis=1))
    oh = toh_ref[:, YD:]
    picked = jnp.sum(logits * oh, axis=1)
    ce_part = jnp.sum(lse - picked)

    t = toh_ref[:, :YD]
    lane = lax.broadcasted_iota(jnp.int32, (TB, NK), 1)
    plo = plo_ref[i]
    pcnt = pcnt_ref[i]

    def pair_step(kk, acc):
        q = plo + kk
        w = w3r_ref[q]
        p = jnp.dot(h2b, w, preferred_element_type=jnp.float32)
        p = p + b3r_ref[q]
        rs_lo = jnp.sum(jnp.where(lane == 2 * q, oh, 0.0), axis=1,
                        keepdims=True)
        rs_hi = jnp.sum(jnp.where(lane == 2 * q + 1, oh, 0.0), axis=1,
                        keepdims=True)
        d_lo = t - p[:, :YD]
        d_hi = t - p[:, YD:]
        return (acc + jnp.sum(d_lo * d_lo * rs_lo)
                + jnp.sum(d_hi * d_hi * rs_hi))

    mse_part = lax.fori_loop(0, pcnt, pair_step, 0.0)
    part = jnp.concatenate([jnp.full((1, 1), ce_part, jnp.float32),
                            jnp.full((1, 1), mse_part, jnp.float32)], axis=1)
    prev = jnp.where(i == 0, jnp.zeros((1, 2), jnp.float32), acc_ref[...])
    acc_ref[...] = prev + part


def _sc_gather_body(x_hbm, toh_hbm, sidx_hbm,
                    g1_hbm, g2_hbm,
                    idx_v, b1a, b1b, b2a, b2b, sga, sgb, swa, swb):
    wid = lax.axis_index("s") * _NC + lax.axis_index("c")
    base = wid * _BPW
    pltpu.sync_copy(sidx_hbm.at[pl.ds(base, _BPW)], idx_v)

    bufs1 = (b1a, b1b)
    bufs2 = (b2a, b2b)
    gsems = (sga, sgb)
    wsems = (swa, swb)
    writes = [None, None]
    for ci in range(_NCH):
        bi = ci % 2
        if writes[bi] is not None:
            for w in writes[bi]:
                w.wait()
        idx_c = idx_v.at[pl.ds(ci * _CH, _CH)]
        c1 = pltpu.async_copy(x_hbm.at[idx_c], bufs1[bi], gsems[bi])
        c2 = pltpu.async_copy(toh_hbm.at[idx_c], bufs2[bi], gsems[bi])
        c1.wait()
        c2.wait()
        dst = pl.ds(base + ci * _CH, _CH)
        w1 = pltpu.async_copy(bufs1[bi], g1_hbm.at[dst], wsems[bi])
        w2 = pltpu.async_copy(bufs2[bi], g2_hbm.at[dst], wsems[bi])
        writes[bi] = (w1, w2)
    for ws in writes:
        if ws is not None:
            for w in ws:
                w.wait()


def _sc_gather(x_batch, toh, sidx):
    mesh = plsc.VectorSubcoreMesh(core_axis_name="c", subcore_axis_name="s",
                                  num_cores=_NC, num_subcores=_NS)
    f = pl.kernel(
        _sc_gather_body,
        out_type=[
            jax.ShapeDtypeStruct((B, XD), jnp.float32),
            jax.ShapeDtypeStruct((B, YD + NK), jnp.float32),
        ],
        mesh=mesh,
        scratch_types=[
            pltpu.VMEM((_BPW,), jnp.int32),
            pltpu.VMEM((_CH, XD), jnp.float32),
            pltpu.VMEM((_CH, XD), jnp.float32),
            pltpu.VMEM((_CH, YD + NK), jnp.float32),
            pltpu.VMEM((_CH, YD + NK), jnp.float32),
            pltpu.SemaphoreType.DMA,
            pltpu.SemaphoreType.DMA,
            pltpu.SemaphoreType.DMA,
            pltpu.SemaphoreType.DMA,
        ],
    )
    return f(x_batch, toh, sidx)


def _compute_labels(y_batch, centers):
    return pl.pallas_call(
        _labels_body,
        grid=(NB,),
        in_specs=[
            pl.BlockSpec((TB, YD), lambda i: (i, 0)),
            pl.BlockSpec((NK, YD), lambda i: (0, 0)),
        ],
        out_specs=[
            pl.BlockSpec((TB, NK), lambda i: (i, 0)),
            pl.BlockSpec((TB, YD + NK), lambda i: (i, 0)),
        ],
        out_shape=[
            jax.ShapeDtypeStruct((B, NK), jnp.int32),
            jax.ShapeDtypeStruct((B, YD + NK), jnp.float32),
        ],
    )(y_batch, centers)


def _fused_loss(plo, pcnt, g1, w1x, b1, w2, b2, w3l, b3l, w3rp, b3rp, g2):
    grid_spec = pltpu.PrefetchScalarGridSpec(
        num_scalar_prefetch=2,
        grid=(NB,),
        in_specs=[
            pl.BlockSpec((TB, XD), lambda i, plo, pcnt: (i, 0)),
            pl.BlockSpec((XD, HID), lambda i, plo, pcnt: (0, 0)),
            pl.BlockSpec((1, HID), lambda i, plo, pcnt: (0, 0)),
            pl.BlockSpec((HID, HID), lambda i, plo, pcnt: (0, 0)),
            pl.BlockSpec((1, HID), lambda i, plo, pcnt: (0, 0)),
            pl.BlockSpec((HID, NK), lambda i, plo, pcnt: (0, 0)),
            pl.BlockSpec((1, NK), lambda i, plo, pcnt: (0, 0)),
            pl.BlockSpec((NP, HID, 2 * YD), lambda i, plo, pcnt: (0, 0, 0)),
            pl.BlockSpec((NP, 1, 2 * YD), lambda i, plo, pcnt: (0, 0, 0)),
            pl.BlockSpec((TB, YD + NK), lambda i, plo, pcnt: (i, 0)),
        ],
        out_specs=pl.BlockSpec((1, 2), lambda i, plo, pcnt: (0, 0)),
    )
    return pl.pallas_call(
        _fused_body,
        grid_spec=grid_spec,
        out_shape=jax.ShapeDtypeStruct((1, 2), jnp.float32),
    )(plo, pcnt, g1, w1x, b1, w2, b2, w3l, b3l, w3rp, b3rp, g2)


def kernel(x_batch, y_batch, W1, b1, W2, b2, W3, b3, centers):
    lab_full, toh = _compute_labels(y_batch, centers)
    labels = lab_full[:, 0]

    slab, sidx = lax.sort_key_val(labels, jnp.arange(B, dtype=jnp.int32))
    blk = slab.reshape(NB, TB)
    plo = (blk[:, 0] // 2).astype(jnp.int32)
    pcnt = (blk[:, -1] // 2 - blk[:, 0] // 2 + 1).astype(jnp.int32)

    g1, g2 = _sc_gather(x_batch, toh, sidx)

    w1x = W1[YD:YD + XD, :].astype(jnp.bfloat16)
    w2 = W2.astype(jnp.bfloat16)
    w3l = W3[:, :NK].astype(jnp.bfloat16)
    w3rp = (W3[:, NK:].astype(jnp.bfloat16)
            .reshape(HID, NP, 2 * YD).transpose(1, 0, 2))
    b3rp = b3[NK:].reshape(NP, 1, 2 * YD)
    acc = _fused_loss(plo, pcnt, g1, w1x, b1.reshape(1, HID), w2,
                      b2.reshape(1, HID), w3l, b3[:NK].reshape(1, NK),
                      w3rp, b3rp, g2)

    return acc[0, 0] / B + 100.0 * acc[0, 1] / (B * YD)


# speedup vs baseline: 2.1347x; 1.0430x over previous
import jax
import jax.numpy as jnp
from jax import lax
from jax.experimental import pallas as pl
from jax.experimental.pallas import tpu as pltpu
from jax.experimental.pallas import tpu_sc as plsc

B = 16384
XD = 512
YD = 64
NK = 64
HID = 2048
NP = NK // 2
TB = 256
NB = B // TB
TF = 512
NF = B // TF

_NC = 2
_NS = 16
_NW = _NC * _NS
_BPW = B // _NW
_CH = 64
_NCH = _BPW // _CH


def _labels_body(y_ref, c_ref, lab_ref, toh_ref):
    y = y_ref[...]
    c = c_ref[...]
    d2 = (jnp.sum(y * y, axis=1, keepdims=True)
          - 2.0 * jax.lax.dot_general(y, c, (((1,), (1,)), ((), ())),
                                      preferred_element_type=jnp.float32)
          + jnp.sum(c * c, axis=1)[None, :])
    lab = jnp.argmin(d2, axis=1).astype(jnp.int32)
    lab_ref[...] = jnp.broadcast_to(lab[:, None], (TB, NK))
    lane = lax.broadcasted_iota(jnp.int32, (TB, NK), 1)
    oh = (lane == lab[:, None]).astype(jnp.float32)
    ct = jnp.dot(oh, c, preferred_element_type=jnp.float32)
    toh_ref[...] = jnp.concatenate([y - ct, oh], axis=1)


def _fused_body(plo_ref, pcnt_ref, xs_ref, w1_ref, b1_ref, w2_ref, b2_ref,
                w3l_ref, b3l_ref, w3r_ref, b3r_ref, toh_ref, acc_ref):
    i = pl.program_id(0)
    xs = xs_ref[...].astype(jnp.bfloat16)
    h1 = jnp.dot(xs, w1_ref[...], preferred_element_type=jnp.float32)
    h1 = jnp.maximum(h1 + b1_ref[...], 0.0).astype(jnp.bfloat16)
    h2 = jnp.dot(h1, w2_ref[...], preferred_element_type=jnp.float32)
    h2 = jnp.maximum(h2 + b2_ref[...], 0.0)
    h2b = h2.astype(jnp.bfloat16)
    logits = jnp.dot(h2b, w3l_ref[...], preferred_element_type=jnp.float32)
    logits = logits + b3l_ref[...]
    m = jnp.max(logits, axis=1, keepdims=True)
    lse = m[:, 0] + jnp.log(jnp.sum(jnp.exp(logits - m), axis=1))
    oh = toh_ref[:, YD:]
    picked = jnp.sum(logits * oh, axis=1)
    ce_part = jnp.sum(lse - picked)

    t = toh_ref[:, :YD]
    lane = lax.broadcasted_iota(jnp.int32, (TF, NK), 1)
    plo = plo_ref[i]
    pcnt = pcnt_ref[i]

    def pair_step(kk, acc):
        q = plo + kk
        w = w3r_ref[q]
        p = jnp.dot(h2b, w, preferred_element_type=jnp.float32)
        p = p + b3r_ref[q]
        rs_lo = jnp.sum(jnp.where(lane == 2 * q, oh, 0.0), axis=1,
                        keepdims=True)
        rs_hi = jnp.sum(jnp.where(lane == 2 * q + 1, oh, 0.0), axis=1,
                        keepdims=True)
        d_lo = t - p[:, :YD]
        d_hi = t - p[:, YD:]
        return (acc + jnp.sum(d_lo * d_lo * rs_lo)
                + jnp.sum(d_hi * d_hi * rs_hi))

    mse_part = lax.fori_loop(0, pcnt, pair_step, 0.0)
    part = jnp.concatenate([jnp.full((1, 1), ce_part, jnp.float32),
                            jnp.full((1, 1), mse_part, jnp.float32)], axis=1)
    prev = jnp.where(i == 0, jnp.zeros((1, 2), jnp.float32), acc_ref[...])
    acc_ref[...] = prev + part


def _sc_gather_body(x_hbm, toh_hbm, sidx_hbm,
                    g1_hbm, g2_hbm,
                    idx_v, b1a, b1b, b2a, b2b, sga, sgb, swa, swb):
    wid = lax.axis_index("s") * _NC + lax.axis_index("c")
    base = wid * _BPW
    pltpu.sync_copy(sidx_hbm.at[pl.ds(base, _BPW)], idx_v)

    bufs1 = (b1a, b1b)
    bufs2 = (b2a, b2b)
    gsems = (sga, sgb)
    wsems = (swa, swb)
    writes = [None, None]
    for ci in range(_NCH):
        bi = ci % 2
        if writes[bi] is not None:
            for w in writes[bi]:
                w.wait()
        idx_c = idx_v.at[pl.ds(ci * _CH, _CH)]
        c1 = pltpu.async_copy(x_hbm.at[idx_c], bufs1[bi], gsems[bi])
        c2 = pltpu.async_copy(toh_hbm.at[idx_c], bufs2[bi], gsems[bi])
        c1.wait()
        c2.wait()
        dst = pl.ds(base + ci * _CH, _CH)
        w1 = pltpu.async_copy(bufs1[bi], g1_hbm.at[dst], wsems[bi])
        w2 = pltpu.async_copy(bufs2[bi], g2_hbm.at[dst], wsems[bi])
        writes[bi] = (w1, w2)
    for ws in writes:
        if ws is not None:
            for w in ws:
                w.wait()


def _sc_gather(x_batch, toh, sidx):
    mesh = plsc.VectorSubcoreMesh(core_axis_name="c", subcore_axis_name="s",
                                  num_cores=_NC, num_subcores=_NS)
    f = pl.kernel(
        _sc_gather_body,
        out_type=[
            jax.ShapeDtypeStruct((B, XD), jnp.float32),
            jax.ShapeDtypeStruct((B, YD + NK), jnp.float32),
        ],
        mesh=mesh,
        scratch_types=[
            pltpu.VMEM((_BPW,), jnp.int32),
            pltpu.VMEM((_CH, XD), jnp.float32),
            pltpu.VMEM((_CH, XD), jnp.float32),
            pltpu.VMEM((_CH, YD + NK), jnp.float32),
            pltpu.VMEM((_CH, YD + NK), jnp.float32),
            pltpu.SemaphoreType.DMA,
            pltpu.SemaphoreType.DMA,
            pltpu.SemaphoreType.DMA,
            pltpu.SemaphoreType.DMA,
        ],
    )
    return f(x_batch, toh, sidx)


def _compute_labels(y_batch, centers):
    return pl.pallas_call(
        _labels_body,
        grid=(NB,),
        in_specs=[
            pl.BlockSpec((TB, YD), lambda i: (i, 0)),
            pl.BlockSpec((NK, YD), lambda i: (0, 0)),
        ],
        out_specs=[
            pl.BlockSpec((TB, NK), lambda i: (i, 0)),
            pl.BlockSpec((TB, YD + NK), lambda i: (i, 0)),
        ],
        out_shape=[
            jax.ShapeDtypeStruct((B, NK), jnp.int32),
            jax.ShapeDtypeStruct((B, YD + NK), jnp.float32),
        ],
    )(y_batch, centers)


def _fused_loss(plo, pcnt, g1, w1x, b1, w2, b2, w3l, b3l, w3rp, b3rp, g2):
    grid_spec = pltpu.PrefetchScalarGridSpec(
        num_scalar_prefetch=2,
        grid=(NF,),
        in_specs=[
            pl.BlockSpec((TF, XD), lambda i, plo, pcnt: (i, 0)),
            pl.BlockSpec((XD, HID), lambda i, plo, pcnt: (0, 0)),
            pl.BlockSpec((1, HID), lambda i, plo, pcnt: (0, 0)),
            pl.BlockSpec((HID, HID), lambda i, plo, pcnt: (0, 0)),
            pl.BlockSpec((1, HID), lambda i, plo, pcnt: (0, 0)),
            pl.BlockSpec((HID, NK), lambda i, plo, pcnt: (0, 0)),
            pl.BlockSpec((1, NK), lambda i, plo, pcnt: (0, 0)),
            pl.BlockSpec((NP, HID, 2 * YD), lambda i, plo, pcnt: (0, 0, 0)),
            pl.BlockSpec((NP, 1, 2 * YD), lambda i, plo, pcnt: (0, 0, 0)),
            pl.BlockSpec((TF, YD + NK), lambda i, plo, pcnt: (i, 0)),
        ],
        out_specs=pl.BlockSpec((1, 2), lambda i, plo, pcnt: (0, 0)),
    )
    return pl.pallas_call(
        _fused_body,
        grid_spec=grid_spec,
        out_shape=jax.ShapeDtypeStruct((1, 2), jnp.float32),
    )(plo, pcnt, g1, w1x, b1, w2, b2, w3l, b3l, w3rp, b3rp, g2)


def kernel(x_batch, y_batch, W1, b1, W2, b2, W3, b3, centers):
    lab_full, toh = _compute_labels(y_batch, centers)
    labels = lab_full[:, 0]

    slab, sidx = lax.sort_key_val(labels, jnp.arange(B, dtype=jnp.int32))
    blk = slab.reshape(NF, TF)
    plo = (blk[:, 0] // 2).astype(jnp.int32)
    pcnt = (blk[:, -1] // 2 - blk[:, 0] // 2 + 1).astype(jnp.int32)

    g1, g2 = _sc_gather(x_batch, toh, sidx)

    w1x = W1[YD:YD + XD, :].astype(jnp.bfloat16)
    w2 = W2.astype(jnp.bfloat16)
    w3l = W3[:, :NK].astype(jnp.bfloat16)
    w3rp = (W3[:, NK:].astype(jnp.bfloat16)
            .reshape(HID, NP, 2 * YD).transpose(1, 0, 2))
    b3rp = b3[NK:].reshape(NP, 1, 2 * YD)
    acc = _fused_loss(plo, pcnt, g1, w1x, b1.reshape(1, HID), w2,
                      b2.reshape(1, HID), w3l, b3[:NK].reshape(1, NK),
                      w3rp, b3rp, g2)

    return acc[0, 0] / B + 100.0 * acc[0, 1] / (B * YD)


# speedup vs baseline: 2.1564x; 1.0102x over previous
import jax
import jax.numpy as jnp
from jax import lax
from jax.experimental import pallas as pl
from jax.experimental.pallas import tpu as pltpu
from jax.experimental.pallas import tpu_sc as plsc

B = 16384
XD = 512
YD = 64
NK = 64
HID = 2048
NP = NK // 2
TB = 256
NB = B // TB
TF = 512
NF = B // TF

_NC = 2
_NS = 16
_NW = _NC * _NS
_BPW = B // _NW
_CH = 64
_NCH = _BPW // _CH


def _labels_body(y_ref, c_ref, lab_ref, toh_ref):
    y = y_ref[...]
    c = c_ref[...]
    d2 = (jnp.sum(y * y, axis=1, keepdims=True)
          - 2.0 * jax.lax.dot_general(y, c, (((1,), (1,)), ((), ())),
                                      preferred_element_type=jnp.float32)
          + jnp.sum(c * c, axis=1)[None, :])
    lab = jnp.argmin(d2, axis=1).astype(jnp.int32)
    lab_ref[...] = jnp.broadcast_to(lab[:, None], (TB, NK))
    lane = lax.broadcasted_iota(jnp.int32, (TB, NK), 1)
    oh = (lane == lab[:, None]).astype(jnp.float32)
    ct = jnp.dot(oh, c, preferred_element_type=jnp.float32)
    toh_ref[...] = jnp.concatenate([y - ct, oh], axis=1)


def _fused_body(plo_ref, pcnt_ref, xs_ref, w1_ref, b1_ref, w2_ref, b2_ref,
                w3l_ref, b3l_ref, w3r_ref, b3r_ref, toh_ref, acc_ref):
    i = pl.program_id(0)
    HF = TF // 2
    w1 = w1_ref[...]
    w2 = w2_ref[...]

    def mlp_half(sl):
        xs = xs_ref[sl, :].astype(jnp.bfloat16)
        h1 = jnp.dot(xs, w1, preferred_element_type=jnp.float32)
        h1 = jnp.maximum(h1 + b1_ref[...], 0.0).astype(jnp.bfloat16)
        h2 = jnp.dot(h1, w2, preferred_element_type=jnp.float32)
        return jnp.maximum(h2 + b2_ref[...], 0.0).astype(jnp.bfloat16)

    h2a = mlp_half(pl.ds(0, HF))
    h2b_ = mlp_half(pl.ds(HF, HF))
    h2 = jnp.concatenate([h2a, h2b_], axis=0)

    logits = jnp.dot(h2, w3l_ref[...], preferred_element_type=jnp.float32)
    logits = logits + b3l_ref[...]
    oh = toh_ref[:, YD:]
    t = toh_ref[:, :YD]
    lane = lax.broadcasted_iota(jnp.int32, (TF, NK), 1)
    plo = plo_ref[i]
    pcnt = pcnt_ref[i]

    def pair_term(q, valid):
        w = w3r_ref[q]
        p = jnp.dot(h2, w, preferred_element_type=jnp.float32)
        p = p + b3r_ref[q]
        sel_lo = jnp.where(valid, jnp.where(lane == 2 * q, oh, 0.0), 0.0)
        sel_hi = jnp.where(valid, jnp.where(lane == 2 * q + 1, oh, 0.0), 0.0)
        rs_lo = jnp.sum(sel_lo, axis=1, keepdims=True)
        rs_hi = jnp.sum(sel_hi, axis=1, keepdims=True)
        d_lo = t - p[:, :YD]
        d_hi = t - p[:, YD:]
        return (jnp.sum(d_lo * d_lo * rs_lo)
                + jnp.sum(d_hi * d_hi * rs_hi))

    mse_part = pair_term(plo, True)
    q1 = jnp.minimum(plo + 1, NP - 1)
    mse_part = mse_part + pair_term(q1, jnp.logical_and(pcnt > 1,
                                                        plo + 1 < NP))

    def pair_step(kk, acc):
        return acc + pair_term(plo + kk, True)

    mse_part = lax.fori_loop(2, pcnt, pair_step, mse_part)

    m = jnp.max(logits, axis=1, keepdims=True)
    lse = m[:, 0] + jnp.log(jnp.sum(jnp.exp(logits - m), axis=1))
    picked = jnp.sum(logits * oh, axis=1)
    ce_part = jnp.sum(lse - picked)

    part = jnp.concatenate([jnp.full((1, 1), ce_part, jnp.float32),
                            jnp.full((1, 1), mse_part, jnp.float32)], axis=1)
    prev = jnp.where(i == 0, jnp.zeros((1, 2), jnp.float32), acc_ref[...])
    acc_ref[...] = prev + part


def _sc_gather_body(x_hbm, toh_hbm, sidx_hbm,
                    g1_hbm, g2_hbm,
                    idx_v, b1a, b1b, b2a, b2b, sga, sgb, swa, swb):
    wid = lax.axis_index("s") * _NC + lax.axis_index("c")
    base = wid * _BPW
    pltpu.sync_copy(sidx_hbm.at[pl.ds(base, _BPW)], idx_v)

    bufs1 = (b1a, b1b)
    bufs2 = (b2a, b2b)
    gsems = (sga, sgb)
    wsems = (swa, swb)
    writes = [None, None]
    for ci in range(_NCH):
        bi = ci % 2
        if writes[bi] is not None:
            for w in writes[bi]:
                w.wait()
        idx_c = idx_v.at[pl.ds(ci * _CH, _CH)]
        c1 = pltpu.async_copy(x_hbm.at[idx_c], bufs1[bi], gsems[bi])
        c2 = pltpu.async_copy(toh_hbm.at[idx_c], bufs2[bi], gsems[bi])
        c1.wait()
        c2.wait()
        dst = pl.ds(base + ci * _CH, _CH)
        w1 = pltpu.async_copy(bufs1[bi], g1_hbm.at[dst], wsems[bi])
        w2 = pltpu.async_copy(bufs2[bi], g2_hbm.at[dst], wsems[bi])
        writes[bi] = (w1, w2)
    for ws in writes:
        if ws is not None:
            for w in ws:
                w.wait()


def _sc_gather(x_batch, toh, sidx):
    mesh = plsc.VectorSubcoreMesh(core_axis_name="c", subcore_axis_name="s",
                                  num_cores=_NC, num_subcores=_NS)
    f = pl.kernel(
        _sc_gather_body,
        out_type=[
            jax.ShapeDtypeStruct((B, XD), jnp.float32),
            jax.ShapeDtypeStruct((B, YD + NK), jnp.float32),
        ],
        mesh=mesh,
        scratch_types=[
            pltpu.VMEM((_BPW,), jnp.int32),
            pltpu.VMEM((_CH, XD), jnp.float32),
            pltpu.VMEM((_CH, XD), jnp.float32),
            pltpu.VMEM((_CH, YD + NK), jnp.float32),
            pltpu.VMEM((_CH, YD + NK), jnp.float32),
            pltpu.SemaphoreType.DMA,
            pltpu.SemaphoreType.DMA,
            pltpu.SemaphoreType.DMA,
            pltpu.SemaphoreType.DMA,
        ],
    )
    return f(x_batch, toh, sidx)


def _compute_labels(y_batch, centers):
    return pl.pallas_call(
        _labels_body,
        grid=(NB,),
        in_specs=[
            pl.BlockSpec((TB, YD), lambda i: (i, 0)),
            pl.BlockSpec((NK, YD), lambda i: (0, 0)),
        ],
        out_specs=[
            pl.BlockSpec((TB, NK), lambda i: (i, 0)),
            pl.BlockSpec((TB, YD + NK), lambda i: (i, 0)),
        ],
        out_shape=[
            jax.ShapeDtypeStruct((B, NK), jnp.int32),
            jax.ShapeDtypeStruct((B, YD + NK), jnp.float32),
        ],
    )(y_batch, centers)


def _fused_loss(plo, pcnt, g1, w1x, b1, w2, b2, w3l, b3l, w3rp, b3rp, g2):
    grid_spec = pltpu.PrefetchScalarGridSpec(
        num_scalar_prefetch=2,
        grid=(NF,),
        in_specs=[
            pl.BlockSpec((TF, XD), lambda i, plo, pcnt: (i, 0)),
            pl.BlockSpec((XD, HID), lambda i, plo, pcnt: (0, 0)),
            pl.BlockSpec((1, HID), lambda i, plo, pcnt: (0, 0)),
            pl.BlockSpec((HID, HID), lambda i, plo, pcnt: (0, 0)),
            pl.BlockSpec((1, HID), lambda i, plo, pcnt: (0, 0)),
            pl.BlockSpec((HID, NK), lambda i, plo, pcnt: (0, 0)),
            pl.BlockSpec((1, NK), lambda i, plo, pcnt: (0, 0)),
            pl.BlockSpec((NP, HID, 2 * YD), lambda i, plo, pcnt: (0, 0, 0)),
            pl.BlockSpec((NP, 1, 2 * YD), lambda i, plo, pcnt: (0, 0, 0)),
            pl.BlockSpec((TF, YD + NK), lambda i, plo, pcnt: (i, 0)),
        ],
        out_specs=pl.BlockSpec((1, 2), lambda i, plo, pcnt: (0, 0)),
    )
    return pl.pallas_call(
        _fused_body,
        grid_spec=grid_spec,
        out_shape=jax.ShapeDtypeStruct((1, 2), jnp.float32),
    )(plo, pcnt, g1, w1x, b1, w2, b2, w3l, b3l, w3rp, b3rp, g2)


def kernel(x_batch, y_batch, W1, b1, W2, b2, W3, b3, centers):
    lab_full, toh = _compute_labels(y_batch, centers)
    labels = lab_full[:, 0]

    slab, sidx = lax.sort_key_val(labels, jnp.arange(B, dtype=jnp.int32))
    blk = slab.reshape(NF, TF)
    plo = (blk[:, 0] // 2).astype(jnp.int32)
    pcnt = (blk[:, -1] // 2 - blk[:, 0] // 2 + 1).astype(jnp.int32)

    g1, g2 = _sc_gather(x_batch, toh, sidx)

    w1x = W1[YD:YD + XD, :].astype(jnp.bfloat16)
    w2 = W2.astype(jnp.bfloat16)
    w3l = W3[:, :NK].astype(jnp.bfloat16)
    w3rp = (W3[:, NK:].astype(jnp.bfloat16)
            .reshape(HID, NP, 2 * YD).transpose(1, 0, 2))
    b3rp = b3[NK:].reshape(NP, 1, 2 * YD)
    acc = _fused_loss(plo, pcnt, g1, w1x, b1.reshape(1, HID), w2,
                      b2.reshape(1, HID), w3l, b3[:NK].reshape(1, NK),
                      w3rp, b3rp, g2)

    return acc[0, 0] / B + 100.0 * acc[0, 1] / (B * YD)


# speedup vs baseline: 2.2486x; 1.0428x over previous
import jax
import jax.numpy as jnp
from jax import lax
from jax.experimental import pallas as pl
from jax.experimental.pallas import tpu as pltpu
from jax.experimental.pallas import tpu_sc as plsc

B = 16384
XD = 512
YD = 64
NK = 64
HID = 2048
NP = NK // 2
TB = 256
NB = B // TB
TF = 512
NF = B // TF

_NC = 2
_NS = 16
_NW = _NC * _NS
_BPW = B // _NW
_CH = 64
_NCH = _BPW // _CH


def _labels_body(y_ref, c_ref, lab_ref, toh_ref):
    y = y_ref[...]
    c = c_ref[...]
    d2 = (jnp.sum(y * y, axis=1, keepdims=True)
          - 2.0 * jax.lax.dot_general(y, c, (((1,), (1,)), ((), ())),
                                      preferred_element_type=jnp.float32)
          + jnp.sum(c * c, axis=1)[None, :])
    lab = jnp.argmin(d2, axis=1).astype(jnp.int32)
    lab_ref[...] = jnp.broadcast_to(lab[:, None], (TB, NK))
    lane = lax.broadcasted_iota(jnp.int32, (TB, NK), 1)
    oh = (lane == lab[:, None]).astype(jnp.float32)
    ct = jnp.dot(oh, c, preferred_element_type=jnp.float32)
    toh_ref[...] = jnp.concatenate([y - ct, oh], axis=1)


def _fused_body(plo_ref, pcnt_ref, xs_ref, w1_ref, b1_ref, w2_ref, b2_ref,
                w3l_ref, b3l_ref, w3r_ref, b3r_ref, toh_ref, acc_ref):
    i = pl.program_id(0)
    HF = TF // 2
    w1 = w1_ref[...]
    w2 = w2_ref[...]

    def mlp_half(sl):
        xs = xs_ref[sl, :].astype(jnp.bfloat16)
        h1 = jnp.dot(xs, w1, preferred_element_type=jnp.float32)
        h1 = jnp.maximum(h1 + b1_ref[...], 0.0).astype(jnp.bfloat16)
        h2 = jnp.dot(h1, w2, preferred_element_type=jnp.float32)
        return jnp.maximum(h2 + b2_ref[...], 0.0).astype(jnp.bfloat16)

    h2a = mlp_half(pl.ds(0, HF))
    h2b_ = mlp_half(pl.ds(HF, HF))
    h2 = jnp.concatenate([h2a, h2b_], axis=0)

    logits = jnp.dot(h2, w3l_ref[...], preferred_element_type=jnp.float32)
    logits = logits + b3l_ref[...]
    oh = toh_ref[:, YD:]
    t = toh_ref[:, :YD]
    lane = lax.broadcasted_iota(jnp.int32, (TF, NK), 1)
    plo = plo_ref[i]
    pcnt = pcnt_ref[i]

    def pair_term(q, valid):
        w = w3r_ref[:, pl.ds(q * 2 * YD, 2 * YD)]
        p = jnp.dot(h2, w, preferred_element_type=jnp.float32)
        p = p + b3r_ref[:, pl.ds(q * 2 * YD, 2 * YD)]
        sel_lo = jnp.where(valid, jnp.where(lane == 2 * q, oh, 0.0), 0.0)
        sel_hi = jnp.where(valid, jnp.where(lane == 2 * q + 1, oh, 0.0), 0.0)
        rs_lo = jnp.sum(sel_lo, axis=1, keepdims=True)
        rs_hi = jnp.sum(sel_hi, axis=1, keepdims=True)
        d_lo = t - p[:, :YD]
        d_hi = t - p[:, YD:]
        return (jnp.sum(d_lo * d_lo * rs_lo)
                + jnp.sum(d_hi * d_hi * rs_hi))

    mse_part = pair_term(plo, True)
    q1 = jnp.minimum(plo + 1, NP - 1)
    mse_part = mse_part + pair_term(q1, jnp.logical_and(pcnt > 1,
                                                        plo + 1 < NP))

    def pair_step(kk, acc):
        return acc + pair_term(plo + kk, True)

    mse_part = lax.fori_loop(2, pcnt, pair_step, mse_part)

    m = jnp.max(logits, axis=1, keepdims=True)
    lse = m[:, 0] + jnp.log(jnp.sum(jnp.exp(logits - m), axis=1))
    picked = jnp.sum(logits * oh, axis=1)
    ce_part = jnp.sum(lse - picked)

    part = jnp.concatenate([jnp.full((1, 1), ce_part, jnp.float32),
                            jnp.full((1, 1), mse_part, jnp.float32)], axis=1)
    prev = jnp.where(i == 0, jnp.zeros((1, 2), jnp.float32), acc_ref[...])
    acc_ref[...] = prev + part


def _sc_gather_body(x_hbm, toh_hbm, sidx_hbm,
                    g1_hbm, g2_hbm,
                    idx_v, b1a, b1b, b2a, b2b, sga, sgb, swa, swb):
    wid = lax.axis_index("s") * _NC + lax.axis_index("c")
    base = wid * _BPW
    pltpu.sync_copy(sidx_hbm.at[pl.ds(base, _BPW)], idx_v)

    bufs1 = (b1a, b1b)
    bufs2 = (b2a, b2b)
    gsems = (sga, sgb)
    wsems = (swa, swb)
    writes = [None, None]
    for ci in range(_NCH):
        bi = ci % 2
        if writes[bi] is not None:
            for w in writes[bi]:
                w.wait()
        idx_c = idx_v.at[pl.ds(ci * _CH, _CH)]
        c1 = pltpu.async_copy(x_hbm.at[idx_c], bufs1[bi], gsems[bi])
        c2 = pltpu.async_copy(toh_hbm.at[idx_c], bufs2[bi], gsems[bi])
        c1.wait()
        c2.wait()
        dst = pl.ds(base + ci * _CH, _CH)
        w1 = pltpu.async_copy(bufs1[bi], g1_hbm.at[dst], wsems[bi])
        w2 = pltpu.async_copy(bufs2[bi], g2_hbm.at[dst], wsems[bi])
        writes[bi] = (w1, w2)
    for ws in writes:
        if ws is not None:
            for w in ws:
                w.wait()


def _sc_gather(x_batch, toh, sidx):
    mesh = plsc.VectorSubcoreMesh(core_axis_name="c", subcore_axis_name="s",
                                  num_cores=_NC, num_subcores=_NS)
    f = pl.kernel(
        _sc_gather_body,
        out_type=[
            jax.ShapeDtypeStruct((B, XD), jnp.float32),
            jax.ShapeDtypeStruct((B, YD + NK), jnp.float32),
        ],
        mesh=mesh,
        scratch_types=[
            pltpu.VMEM((_BPW,), jnp.int32),
            pltpu.VMEM((_CH, XD), jnp.float32),
            pltpu.VMEM((_CH, XD), jnp.float32),
            pltpu.VMEM((_CH, YD + NK), jnp.float32),
            pltpu.VMEM((_CH, YD + NK), jnp.float32),
            pltpu.SemaphoreType.DMA,
            pltpu.SemaphoreType.DMA,
            pltpu.SemaphoreType.DMA,
            pltpu.SemaphoreType.DMA,
        ],
    )
    return f(x_batch, toh, sidx)


def _compute_labels(y_batch, centers):
    return pl.pallas_call(
        _labels_body,
        grid=(NB,),
        in_specs=[
            pl.BlockSpec((TB, YD), lambda i: (i, 0)),
            pl.BlockSpec((NK, YD), lambda i: (0, 0)),
        ],
        out_specs=[
            pl.BlockSpec((TB, NK), lambda i: (i, 0)),
            pl.BlockSpec((TB, YD + NK), lambda i: (i, 0)),
        ],
        out_shape=[
            jax.ShapeDtypeStruct((B, NK), jnp.int32),
            jax.ShapeDtypeStruct((B, YD + NK), jnp.float32),
        ],
    )(y_batch, centers)


def _fused_loss(plo, pcnt, g1, w1x, b1, w2, b2, w3l, b3l, w3rp, b3rp, g2):
    grid_spec = pltpu.PrefetchScalarGridSpec(
        num_scalar_prefetch=2,
        grid=(NF,),
        in_specs=[
            pl.BlockSpec((TF, XD), lambda i, plo, pcnt: (i, 0)),
            pl.BlockSpec((XD, HID), lambda i, plo, pcnt: (0, 0)),
            pl.BlockSpec((1, HID), lambda i, plo, pcnt: (0, 0)),
            pl.BlockSpec((HID, HID), lambda i, plo, pcnt: (0, 0)),
            pl.BlockSpec((1, HID), lambda i, plo, pcnt: (0, 0)),
            pl.BlockSpec((HID, NK), lambda i, plo, pcnt: (0, 0)),
            pl.BlockSpec((1, NK), lambda i, plo, pcnt: (0, 0)),
            pl.BlockSpec((HID, NP * 2 * YD), lambda i, plo, pcnt: (0, 0)),
            pl.BlockSpec((1, NP * 2 * YD), lambda i, plo, pcnt: (0, 0)),
            pl.BlockSpec((TF, YD + NK), lambda i, plo, pcnt: (i, 0)),
        ],
        out_specs=pl.BlockSpec((1, 2), lambda i, plo, pcnt: (0, 0)),
    )
    return pl.pallas_call(
        _fused_body,
        grid_spec=grid_spec,
        out_shape=jax.ShapeDtypeStruct((1, 2), jnp.float32),
    )(plo, pcnt, g1, w1x, b1, w2, b2, w3l, b3l, w3rp, b3rp, g2)


def kernel(x_batch, y_batch, W1, b1, W2, b2, W3, b3, centers):
    lab_full, toh = _compute_labels(y_batch, centers)
    labels = lab_full[:, 0]

    slab, sidx = lax.sort_key_val(labels, jnp.arange(B, dtype=jnp.int32))
    blk = slab.reshape(NF, TF)
    plo = (blk[:, 0] // 2).astype(jnp.int32)
    pcnt = (blk[:, -1] // 2 - blk[:, 0] // 2 + 1).astype(jnp.int32)

    g1, g2 = _sc_gather(x_batch, toh, sidx)

    w1x = W1[YD:YD + XD, :].astype(jnp.bfloat16)
    w2 = W2.astype(jnp.bfloat16)
    w3l = W3[:, :NK].astype(jnp.bfloat16)
    w3rp = W3[:, NK:].astype(jnp.bfloat16)
    b3rp = b3[NK:].reshape(1, NP * 2 * YD)
    acc = _fused_loss(plo, pcnt, g1, w1x, b1.reshape(1, HID), w2,
                      b2.reshape(1, HID), w3l, b3[:NK].reshape(1, NK),
                      w3rp, b3rp, g2)

    return acc[0, 0] / B + 100.0 * acc[0, 1] / (B * YD)
